# baseline (speedup 1.0000x reference)
import functools

import jax
import jax.numpy as jnp
from jax import lax
from jax.experimental import pallas as pl
from jax.experimental.pallas import tpu as pltpu


def _s2d(x):
    B, H, W, C = x.shape
    x = x.reshape(B, H // 2, 2, W // 2, 2, C)
    x = x.transpose(0, 1, 3, 2, 4, 5)
    return x.reshape(B, H // 2, W // 2, 4 * C)


def _conv_w(w):
    O, C = w.shape[0], w.shape[1]
    w = w.reshape(O, C, 2, 2, 2, 2)
    w = w.transpose(2, 4, 3, 5, 1, 0)
    return w.reshape(4, 4 * C, O)


def _conv_kernel(s_ref, w_ref, b_ref, o_ref, *, Ho, Wo, relu):
    acc = None
    for dy in range(2):
        for dx in range(2):
            a = s_ref[0, dy:dy + Ho, dx:dx + Wo, :].reshape(Ho * Wo, s_ref.shape[3])
            t = jnp.dot(a, w_ref[2 * dy + dx], preferred_element_type=jnp.float32)
            acc = t if acc is None else acc + t
    acc = acc + b_ref[0]
    if relu:
        acc = jnp.maximum(acc, 0.0)
    o_ref[0] = acc


def _vq_kernel(z_ref, ct_ref, cb_ref, idx_ref, q_ref, *, R, K):
    z = z_ref[0]
    zc = jnp.dot(z, ct_ref[...], preferred_element_type=jnp.float32)
    zsq = jnp.sum(z * z, axis=1, keepdims=True)
    cb = cb_ref[...]
    csq = jnp.sum(cb * cb, axis=1)
    d2 = (zsq - 2.0 * zc) + csq[None, :]
    idx = jnp.argmin(d2, axis=1).astype(jnp.int32)
    onehot = (idx[:, None] == lax.broadcasted_iota(jnp.int32, (R, K), 1)).astype(jnp.float32)
    q = jnp.dot(onehot, cb, preferred_element_type=jnp.float32)
    idx_ref[0, 0] = idx
    q_ref[0] = q


def _deconv_kernel(h_ref, w_ref, b_ref, o_ref, *, Ho, Wo, relu):
    parts = []
    for py in range(2):
        for px in range(2):
            acc = None
            for dy in range(2):
                for dx in range(2):
                    a = h_ref[0, py + dy:py + dy + Ho, px + dx:px + dx + Wo, :]
                    a = a.reshape(Ho * Wo, h_ref.shape[3])
                    w = w_ref[2 * dy + py, 2 * dx + px]
                    t = jnp.dot(a, w, preferred_element_type=jnp.float32)
                    acc = t if acc is None else acc + t
            acc = acc + b_ref[0]
            if relu:
                acc = jnp.maximum(acc, 0.0)
            parts.append(acc)
    o_ref[0] = jnp.concatenate(parts, axis=1)


def _run_conv(s, wm, b, Ho, Wo, relu):
    B, Hs, Ws, F = s.shape
    O = wm.shape[2]
    return pl.pallas_call(
        functools.partial(_conv_kernel, Ho=Ho, Wo=Wo, relu=relu),
        grid=(B,),
        in_specs=[
            pl.BlockSpec((1, Hs, Ws, F), lambda i: (i, 0, 0, 0)),
            pl.BlockSpec((4, F, O), lambda i: (0, 0, 0)),
            pl.BlockSpec((1, O), lambda i: (0, 0)),
        ],
        out_specs=pl.BlockSpec((1, Ho * Wo, O), lambda i: (i, 0, 0)),
        out_shape=jax.ShapeDtypeStruct((B, Ho * Wo, O), jnp.float32),
    )(s, wm, b.reshape(1, O))


def _run_deconv(h, wt, b, Ho, Wo, relu):
    B, Hs, Ws, Ci = h.shape
    Co = wt.shape[3]
    return pl.pallas_call(
        functools.partial(_deconv_kernel, Ho=Ho, Wo=Wo, relu=relu),
        grid=(B,),
        in_specs=[
            pl.BlockSpec((1, Hs, Ws, Ci), lambda i: (i, 0, 0, 0)),
            pl.BlockSpec((4, 4, Ci, Co), lambda i: (0, 0, 0, 0)),
            pl.BlockSpec((1, Co), lambda i: (0, 0)),
        ],
        out_specs=pl.BlockSpec((1, Ho * Wo, 4 * Co), lambda i: (i, 0, 0)),
        out_shape=jax.ShapeDtypeStruct((B, Ho * Wo, 4 * Co), jnp.float32),
    )(h, wt, b.reshape(1, Co))


def kernel(x, x_cond, y, enc_w1, enc_b1, enc_w2, enc_b2, codebook,
           dec_w1, dec_b1, dec_w2, dec_b2):
    B = x.shape[0]
    K, D = codebook.shape

    xt = x.transpose(0, 2, 3, 1)
    xp = jnp.pad(xt, ((0, 0), (1, 1), (1, 1), (0, 0)))
    s1 = _s2d(xp)
    h1 = _run_conv(s1, _conv_w(enc_w1), enc_b1, 112, 112, relu=True)

    h1sp = h1.reshape(B, 112, 112, 64)
    h1p = jnp.pad(h1sp, ((0, 0), (1, 1), (1, 1), (0, 0)))
    s2 = _s2d(h1p)
    lat = _run_conv(s2, _conv_w(enc_w2), enc_b2, 56, 56, relu=False)

    R = 448
    T = (56 * 56) // R
    ct = codebook.T.copy()
    idx_r, q = pl.pallas_call(
        functools.partial(_vq_kernel, R=R, K=K),
        grid=(B, T),
        in_specs=[
            pl.BlockSpec((1, R, D), lambda b, t: (b, t, 0)),
            pl.BlockSpec((D, K), lambda b, t: (0, 0)),
            pl.BlockSpec((K, D), lambda b, t: (0, 0)),
        ],
        out_specs=[
            pl.BlockSpec((1, 1, R), lambda b, t: (b * T + t, 0, 0)),
            pl.BlockSpec((1, R, D), lambda b, t: (b, t, 0)),
        ],
        out_shape=[
            jax.ShapeDtypeStruct((B * T, 1, R), jnp.int32),
            jax.ShapeDtypeStruct((B, 56 * 56, D), jnp.float32),
        ],
    )(lat.reshape(B, T, R, D).reshape(B, 56 * 56, D), ct, codebook)
    idx = idx_r.reshape(B, 56, 56)

    qsp = q.reshape(B, 56, 56, D)
    qpad = jnp.pad(qsp, ((0, 0), (1, 1), (1, 1), (0, 0)))
    wt1 = dec_w1.transpose(2, 3, 1, 0)
    o1 = _run_deconv(qpad, wt1, dec_b1, 56, 56, relu=True)
    h2 = o1.reshape(B, 56, 56, 2, 2, 64).transpose(0, 1, 3, 2, 4, 5).reshape(B, 112, 112, 64)

    h2p = jnp.pad(h2, ((0, 0), (1, 1), (1, 1), (0, 0)))
    wt2 = dec_w2.transpose(2, 3, 1, 0)
    o2 = _run_deconv(h2p, wt2, dec_b2, 112, 112, relu=False)
    xhat = (o2.reshape(B, 112, 112, 2, 2, 3).transpose(0, 1, 3, 2, 4, 5)
            .reshape(B, 224, 224, 3).transpose(0, 3, 1, 2))

    latent = lat.reshape(B, 56, 56, D).transpose(0, 3, 1, 2)
    quantized = qsp.transpose(0, 3, 1, 2)
    return (xhat, quantized, latent, idx)
